# Initial kernel scaffold; baseline (speedup 1.0000x reference)
import jax
import jax.numpy as jnp
from jax.experimental import pallas as pl


def kernel(x, qr, mask, W_qb, W_k, ln_g, ln_b, W_w):
    raise NotImplementedError("write your pallas kernel here")


# speedup vs baseline: 1.2660x; 1.2660x over previous
import functools
import math

import jax
import jax.numpy as jnp
from jax.experimental import pallas as pl
from jax.experimental.pallas import tpu as pltpu

_N_HEADS = 32
_HEAD_DIM = 128
_ROPE_DIM = 64
_TOPK = 2048
_BASE = 10000.0
_EPS = 1e-5


def _rope_tables(s: int):
    pos = jnp.arange(s, dtype=jnp.float32)
    freqs = _BASE ** (-jnp.arange(0, _ROPE_DIM, 2, dtype=jnp.float32) / _ROPE_DIM)
    theta = pos[:, None] * freqs[None, :]
    cos = jnp.cos(theta)
    sin = jnp.sin(theta)
    cc = jnp.repeat(cos, 2, axis=1)
    ss_even = -sin
    ss_odd = sin
    ss = jnp.stack([ss_even, ss_odd], axis=-1).reshape(s, _ROPE_DIM)
    ones = jnp.ones((s, _HEAD_DIM - _ROPE_DIM), jnp.float32)
    zeros = jnp.zeros((s, _HEAD_DIM - _ROPE_DIM), jnp.float32)
    CC = jnp.concatenate([cc, ones], axis=1)
    SS = jnp.concatenate([ss, zeros], axis=1)
    i = jnp.arange(_HEAD_DIM)
    j = jnp.arange(_HEAD_DIM)
    swap = (i[:, None] // 2 == j[None, :] // 2) & (i[:, None] != j[None, :])
    P = jnp.where((i[:, None] < _ROPE_DIM) & swap, 1.0, 0.0).astype(jnp.float32)
    return CC, SS, P


def _prologue(x_ref, wk_ref, ww_ref, lng_ref, lnb_ref, cc_ref, ss_ref, p_ref,
              k_ref, w_ref, *, w_scale):
    x_b = x_ref[...].astype(jnp.bfloat16)
    kx = jnp.dot(x_b, wk_ref[...].astype(jnp.bfloat16),
                 preferred_element_type=jnp.float32)
    m = jnp.mean(kx, axis=-1, keepdims=True)
    c = kx - m
    v = jnp.mean(c * c, axis=-1, keepdims=True)
    normed = c / jnp.sqrt(v + _EPS) * lng_ref[...] + lnb_ref[...]
    k_rot = (normed * cc_ref[...]
             + jnp.dot(normed, p_ref[...], preferred_element_type=jnp.float32, precision=jax.lax.Precision.HIGHEST)
             * ss_ref[...])
    k_ref[...] = k_rot
    w_ref[...] = jnp.dot(x_b, ww_ref[...].astype(jnp.bfloat16),
                         preferred_element_type=jnp.float32) * w_scale


def _scores(qr_ref, wqb_ref, k_ref, w_ref, cc_ref, ss_ref, p_ref, out_ref):
    qfull = jnp.dot(qr_ref[...].astype(jnp.bfloat16),
                    wqb_ref[...].astype(jnp.bfloat16),
                    preferred_element_type=jnp.float32)
    cc = cc_ref[...]
    ss = ss_ref[...]
    p = p_ref[...]
    k = k_ref[...].astype(jnp.bfloat16)
    acc = None
    for h in range(_N_HEADS):
        qh = qfull[:, h * _HEAD_DIM:(h + 1) * _HEAD_DIM]
        qh_rot = qh * cc + jnp.dot(qh, p, preferred_element_type=jnp.float32, precision=jax.lax.Precision.HIGHEST) * ss
        sc = jax.lax.dot_general(qh_rot.astype(jnp.bfloat16), k,
                                 (((1,), (1,)), ((), ())),
                                 preferred_element_type=jnp.float32)
        term = jnp.maximum(sc, 0.0) * w_ref[:, h:h + 1]
        acc = term if acc is None else acc + term
    out_ref[...] = acc


def _compute_scores(x, qr, W_qb, W_k, ln_g, ln_b, W_w):
    s = x.shape[0]
    CC, SS, P = _rope_tables(s)
    w_scale = (_N_HEADS ** -0.5) * (_HEAD_DIM ** -0.5)

    KBLK = 512
    k_rot, w = pl.pallas_call(
        functools.partial(_prologue, w_scale=w_scale),
        grid=(s // KBLK,),
        in_specs=[
            pl.BlockSpec((KBLK, x.shape[1]), lambda i: (i, 0)),
            pl.BlockSpec((x.shape[1], _HEAD_DIM), lambda i: (0, 0)),
            pl.BlockSpec((x.shape[1], _N_HEADS), lambda i: (0, 0)),
            pl.BlockSpec((1, _HEAD_DIM), lambda i: (0, 0)),
            pl.BlockSpec((1, _HEAD_DIM), lambda i: (0, 0)),
            pl.BlockSpec((KBLK, _HEAD_DIM), lambda i: (i, 0)),
            pl.BlockSpec((KBLK, _HEAD_DIM), lambda i: (i, 0)),
            pl.BlockSpec((_HEAD_DIM, _HEAD_DIM), lambda i: (0, 0)),
        ],
        out_specs=[
            pl.BlockSpec((KBLK, _HEAD_DIM), lambda i: (i, 0)),
            pl.BlockSpec((KBLK, _N_HEADS), lambda i: (i, 0)),
        ],
        out_shape=[
            jax.ShapeDtypeStruct((s, _HEAD_DIM), jnp.float32),
            jax.ShapeDtypeStruct((s, _N_HEADS), jnp.float32),
        ],
    )(x, W_k, W_w, ln_g.reshape(1, -1), ln_b.reshape(1, -1), CC, SS, P)

    QBLK = 256
    scores = pl.pallas_call(
        _scores,
        grid=(s // QBLK,),
        in_specs=[
            pl.BlockSpec((QBLK, qr.shape[1]), lambda i: (i, 0)),
            pl.BlockSpec((qr.shape[1], _N_HEADS * _HEAD_DIM), lambda i: (0, 0)),
            pl.BlockSpec((s, _HEAD_DIM), lambda i: (0, 0)),
            pl.BlockSpec((QBLK, _N_HEADS), lambda i: (i, 0)),
            pl.BlockSpec((QBLK, _HEAD_DIM), lambda i: (i, 0)),
            pl.BlockSpec((QBLK, _HEAD_DIM), lambda i: (i, 0)),
            pl.BlockSpec((_HEAD_DIM, _HEAD_DIM), lambda i: (0, 0)),
        ],
        out_specs=pl.BlockSpec((QBLK, s), lambda i: (i, 0)),
        out_shape=jax.ShapeDtypeStruct((s, s), jnp.float32),
        compiler_params=pltpu.CompilerParams(
            dimension_semantics=("arbitrary",),
        ),
    )(qr, W_qb, k_rot, w, CC, SS, P)
    return scores


def kernel(x, qr, mask, W_qb, W_k, ln_g, ln_b, W_w):
    b, s, _ = x.shape
    scores = _compute_scores(x[0], qr[0], W_qb, W_k, ln_g, ln_b, W_w)
    _, idx = jax.lax.top_k(scores[None, None], _TOPK)
    return idx
